# baseline (speedup 1.0000x reference)
import functools

import jax
import jax.numpy as jnp
from jax import lax
from jax.experimental import pallas as pl
from jax.experimental.pallas import tpu as pltpu
from jax.experimental.pallas import tpu_sc as plsc

GAMMA = 12.0
BATCH = 16384
HIDDEN = 128
LANES = 16

NUM_CORES = 2
NUM_SUBCORES = 16
NUM_WORKERS = NUM_CORES * NUM_SUBCORES
B_PER_W = BATCH // NUM_WORKERS
CHUNK = 16
N_CHUNKS = B_PER_W // CHUNK
GROUPS = CHUNK // LANES
DIM_CHUNKS = HIDDEN // LANES
NBUF = 8


def _tree_sum(vals):
    vals = list(vals)
    while len(vals) > 1:
        nxt = [a + b for a, b in zip(vals[::2], vals[1::2])]
        if len(vals) % 2:
            nxt.append(vals[-1])
        vals = nxt
    return vals[0]


def _make_kernel():
    mesh = plsc.VectorSubcoreMesh(core_axis_name="c", subcore_axis_name="s")

    @functools.partial(
        pl.kernel,
        mesh=mesh,
        out_type=jax.ShapeDtypeStruct((BATCH, 1), jnp.float32),
        compiler_params=pltpu.CompilerParams(needs_layout_passes=False),
        scratch_types=[
            pltpu.VMEM((B_PER_W,), jnp.int32),
            pltpu.VMEM((B_PER_W,), jnp.int32),
            pltpu.VMEM((B_PER_W,), jnp.int32),
            pltpu.VMEM((NBUF, CHUNK, HIDDEN), jnp.float32),
            pltpu.VMEM((NBUF, CHUNK, HIDDEN), jnp.float32),
            pltpu.VMEM((NBUF, CHUNK, HIDDEN), jnp.float32),
            pltpu.VMEM((LANES * LANES,), jnp.float32),
            pltpu.VMEM((B_PER_W, 1), jnp.float32),
            pltpu.SemaphoreType.DMA((NBUF,)),
            pltpu.SemaphoreType.DMA((NBUF,)),
            pltpu.SemaphoreType.DMA((NBUF,)),
            pltpu.SemaphoreType.DMA,
        ],
    )
    def kge_score(h_idx_hbm, r_idx_hbm, t_idx_hbm, ent_hbm, rel_hbm, out_hbm,
                  hidx, ridx, tidx, h_bufs, r_bufs, t_bufs, p_mat, out_all,
                  sem_h, sem_r, sem_t, sem_i):
        wid = lax.axis_index("s") * NUM_CORES + lax.axis_index("c")
        w_base = wid * B_PER_W
        row_ids = lax.iota(jnp.int32, LANES)
        zeros16 = jnp.zeros((LANES,), jnp.int32)

        cp1 = pltpu.async_copy(
            h_idx_hbm.at[pl.ds(w_base, B_PER_W)], hidx, sem_i)
        cp2 = pltpu.async_copy(
            r_idx_hbm.at[pl.ds(w_base, B_PER_W)], ridx, sem_i)
        cp3 = pltpu.async_copy(
            t_idx_hbm.at[pl.ds(w_base, B_PER_W)], tidx, sem_i)
        cp1.wait()
        cp2.wait()
        cp3.wait()

        def start_gathers(cc):
            b = lax.rem(cc, NBUF)
            sl = pl.ds(cc * CHUNK, CHUNK)
            pltpu.async_copy(ent_hbm.at[hidx.at[sl]], h_bufs.at[b],
                             sem_h.at[b])
            pltpu.async_copy(rel_hbm.at[ridx.at[sl]], r_bufs.at[b],
                             sem_r.at[b])
            pltpu.async_copy(ent_hbm.at[tidx.at[sl]], t_bufs.at[b],
                             sem_t.at[b])

        def wait_gathers(b):
            sl = pl.ds(0, CHUNK)
            pltpu.make_async_copy(ent_hbm.at[hidx.at[sl]], h_bufs.at[b],
                                  sem_h.at[b]).wait()
            pltpu.make_async_copy(rel_hbm.at[ridx.at[sl]], r_bufs.at[b],
                                  sem_r.at[b]).wait()
            pltpu.make_async_copy(ent_hbm.at[tidx.at[sl]], t_bufs.at[b],
                                  sem_t.at[b]).wait()

        for cc in range(NBUF - 1):
            start_gathers(cc)

        @pl.loop(0, N_CHUNKS)
        def _chunk(c):
            b = lax.rem(c, NBUF)

            @pl.when(c + NBUF - 1 < N_CHUNKS)
            def _():
                start_gathers(c + NBUF - 1)

            wait_gathers(b)

            for g in range(GROUPS):
                for s in range(LANES):
                    row = g * LANES + s
                    terms = []
                    for k in range(DIM_CHUNKS):
                        h = h_bufs[b, row, pl.ds(k * LANES, LANES)]
                        t = t_bufs[b, row, pl.ds(k * LANES, LANES)]
                        r = r_bufs[b, row, pl.ds(k * LANES, LANES)]
                        terms.append(jnp.abs(h + (r - t)))
                    p_mat[pl.ds(s * LANES, LANES)] = _tree_sum(terms)
                cols = [plsc.load_gather(p_mat, [row_ids * LANES + j])
                        for j in range(LANES)]
                plsc.store_scatter(out_all,
                                   [c * CHUNK + g * LANES + row_ids, zeros16],
                                   jnp.float32(GAMMA) - _tree_sum(cols))

        pltpu.sync_copy(out_all, out_hbm.at[pl.ds(w_base, B_PER_W), :])

    return kge_score


_KGE_KERNEL = _make_kernel()


def kernel(sample, entity_embedding, relation_embedding):
    h_idx = sample[:, 0]
    r_idx = sample[:, 1]
    t_idx = sample[:, 2]
    return _KGE_KERNEL(h_idx, r_idx, t_idx, entity_embedding,
                       relation_embedding)


# speedup vs baseline: 1.2280x; 1.2280x over previous
import functools

import jax
import jax.numpy as jnp
from jax import lax
from jax.experimental import pallas as pl
from jax.experimental.pallas import tpu as pltpu
from jax.experimental.pallas import tpu_sc as plsc

GAMMA = 12.0
BATCH = 16384
HIDDEN = 128
LANES = 16

NUM_CORES = 2
NUM_SUBCORES = 16
NUM_WORKERS = NUM_CORES * NUM_SUBCORES
B_PER_W = BATCH // NUM_WORKERS
CHUNK = 16
N_CHUNKS = B_PER_W // CHUNK
GROUPS = CHUNK // LANES
DIM_CHUNKS = HIDDEN // LANES
NBUF = 8


def _tree_sum(vals):
    vals = list(vals)
    while len(vals) > 1:
        nxt = [a + b for a, b in zip(vals[::2], vals[1::2])]
        if len(vals) % 2:
            nxt.append(vals[-1])
        vals = nxt
    return vals[0]


def _make_kernel():
    mesh = plsc.VectorSubcoreMesh(core_axis_name="c", subcore_axis_name="s")

    @functools.partial(
        pl.kernel,
        mesh=mesh,
        out_type=jax.ShapeDtypeStruct((BATCH,), jnp.float32),
        compiler_params=pltpu.CompilerParams(needs_layout_passes=False),
        scratch_types=[
            pltpu.VMEM((B_PER_W,), jnp.int32),
            pltpu.VMEM((B_PER_W,), jnp.int32),
            pltpu.VMEM((B_PER_W,), jnp.int32),
            pltpu.VMEM((NBUF, CHUNK, HIDDEN), jnp.float32),
            pltpu.VMEM((NBUF, CHUNK, HIDDEN), jnp.float32),
            pltpu.VMEM((NBUF, CHUNK, HIDDEN), jnp.float32),
            pltpu.VMEM((LANES * LANES,), jnp.float32),
            pltpu.VMEM((B_PER_W,), jnp.float32),
            pltpu.SemaphoreType.DMA((NBUF,)),
            pltpu.SemaphoreType.DMA((NBUF,)),
            pltpu.SemaphoreType.DMA((NBUF,)),
            pltpu.SemaphoreType.DMA,
        ],
    )
    def kge_score(h_idx_hbm, r_idx_hbm, t_idx_hbm, ent_hbm, rel_hbm, out_hbm,
                  hidx, ridx, tidx, h_bufs, r_bufs, t_bufs, p_mat, out_all,
                  sem_h, sem_r, sem_t, sem_i):
        wid = lax.axis_index("s") * NUM_CORES + lax.axis_index("c")
        w_base = wid * B_PER_W
        row_ids = lax.iota(jnp.int32, LANES)

        cp1 = pltpu.async_copy(
            h_idx_hbm.at[pl.ds(w_base, B_PER_W)], hidx, sem_i)
        cp2 = pltpu.async_copy(
            r_idx_hbm.at[pl.ds(w_base, B_PER_W)], ridx, sem_i)
        cp3 = pltpu.async_copy(
            t_idx_hbm.at[pl.ds(w_base, B_PER_W)], tidx, sem_i)
        cp1.wait()
        cp2.wait()
        cp3.wait()

        def start_gathers(cc):
            b = lax.rem(cc, NBUF)
            sl = pl.ds(cc * CHUNK, CHUNK)
            pltpu.async_copy(ent_hbm.at[hidx.at[sl]], h_bufs.at[b],
                             sem_h.at[b])
            pltpu.async_copy(rel_hbm.at[ridx.at[sl]], r_bufs.at[b],
                             sem_r.at[b])
            pltpu.async_copy(ent_hbm.at[tidx.at[sl]], t_bufs.at[b],
                             sem_t.at[b])

        def wait_gathers(b):
            sl = pl.ds(0, CHUNK)
            pltpu.make_async_copy(ent_hbm.at[hidx.at[sl]], h_bufs.at[b],
                                  sem_h.at[b]).wait()
            pltpu.make_async_copy(rel_hbm.at[ridx.at[sl]], r_bufs.at[b],
                                  sem_r.at[b]).wait()
            pltpu.make_async_copy(ent_hbm.at[tidx.at[sl]], t_bufs.at[b],
                                  sem_t.at[b]).wait()

        for cc in range(NBUF - 1):
            start_gathers(cc)

        @pl.loop(0, N_CHUNKS)
        def _chunk(c):
            b = lax.rem(c, NBUF)

            @pl.when(c + NBUF - 1 < N_CHUNKS)
            def _():
                start_gathers(c + NBUF - 1)

            wait_gathers(b)

            for g in range(GROUPS):
                for s in range(LANES):
                    row = g * LANES + s
                    terms = []
                    for k in range(DIM_CHUNKS):
                        h = h_bufs[b, row, pl.ds(k * LANES, LANES)]
                        t = t_bufs[b, row, pl.ds(k * LANES, LANES)]
                        r = r_bufs[b, row, pl.ds(k * LANES, LANES)]
                        terms.append(jnp.abs(h + (r - t)))
                    p_mat[pl.ds(s * LANES, LANES)] = _tree_sum(terms)
                cols = [plsc.load_gather(p_mat, [row_ids * LANES + j])
                        for j in range(LANES)]
                out_all[pl.ds(c * CHUNK + g * LANES, LANES)] = (
                    jnp.float32(GAMMA) - _tree_sum(cols))

        pltpu.sync_copy(out_all, out_hbm.at[pl.ds(w_base, B_PER_W)])

    return kge_score


_KGE_KERNEL = _make_kernel()


def kernel(sample, entity_embedding, relation_embedding):
    h_idx = sample[:, 0]
    r_idx = sample[:, 1]
    t_idx = sample[:, 2]
    scores = _KGE_KERNEL(h_idx, r_idx, t_idx, entity_embedding,
                         relation_embedding)
    return scores.reshape(BATCH, 1)
